# baseline (speedup 1.0000x reference)
import functools

import jax
import jax.numpy as jnp
from jax import lax
from jax.experimental import pallas as pl
from jax.experimental.pallas import tpu as pltpu
from jax.experimental.pallas import tpu_sc as plsc

N = 5000
NP = 5120
BLK = 128
NB = NP // BLK
CHUNK = 1024
NCHUNK = NP // CHUNK
THR = 0.45
F32 = jnp.float32


def _ident():
    r = lax.broadcasted_iota(jnp.int32, (BLK, BLK), 0)
    c = lax.broadcasted_iota(jnp.int32, (BLK, BLK), 1)
    return (r == c).astype(F32)


def _row_to_col(v):
    return jnp.sum(_ident() * v, axis=1, keepdims=True)


def _col_to_row(u):
    return jnp.sum(_ident() * u, axis=0, keepdims=True)


def _rank_kernel(s_ref, rank_ref):
    srow = s_ref[0:1, :]
    cid = lax.broadcasted_iota(jnp.int32, (1, NP), 1)

    def body(bi, _):
        base = bi * BLK
        s_blk = s_ref[0:1, pl.ds(base, BLK)]
        s_col = _row_to_col(s_blk)
        rid = base + lax.broadcasted_iota(jnp.int32, (BLK, 1), 0)
        before = (srow > s_col) | ((srow == s_col) & (cid < rid))
        cnt = jnp.sum(before.astype(jnp.int32), axis=1, keepdims=True)
        rank_ref[pl.ds(base, BLK), :] = cnt
        return 0

    lax.fori_loop(0, NB, body, 0)


def _rank_call(s_row):
    return pl.pallas_call(
        _rank_kernel,
        out_shape=jax.ShapeDtypeStruct((NP, 1), jnp.int32),
    )(s_row)


_NC, _NS = 2, 16
_NW = _NC * _NS
_RPW = NP // _NW
_HALF = _RPW // 2
_D = 128


@functools.cache
def _make_scatter_sc():
    @functools.partial(
        pl.kernel,
        out_type=jax.ShapeDtypeStruct((NP, _D), F32),
        mesh=plsc.VectorSubcoreMesh(core_axis_name="c", subcore_axis_name="s"),
        scratch_types=[
            pltpu.VMEM((2, _HALF), jnp.int32),
            pltpu.VMEM((_RPW, _D), F32),
            pltpu.SemaphoreType.DMA,
        ],
    )
    def _scatter_sc(data_hbm, rank_hbm, out_hbm, idx_v, rows_v, sem):
        wid = lax.axis_index("s") * _NC + lax.axis_index("c")
        base = wid * _RPW
        pltpu.sync_copy(rank_hbm.at[wid], idx_v)
        pltpu.sync_copy(data_hbm.at[pl.ds(base, _RPW)], rows_v)
        for h in range(2):
            pltpu.async_copy(
                rows_v.at[pl.ds(h * _HALF, _HALF)],
                out_hbm.at[idx_v.at[h]],
                sem,
            ).wait()

    return _scatter_sc


def _gather_call_sc(data, rank):
    rank3 = rank[:, 0].reshape(_NW, 2, _HALF)
    return _make_scatter_sc()(data, rank3)


def _iou(x1c, y1c, x2c, y2c, ac, x1r, y1r, x2r, y2r, ar):
    xx1 = jnp.maximum(x1c, x1r)
    yy1 = jnp.maximum(y1c, y1r)
    xx2 = jnp.minimum(x2c, x2r)
    yy2 = jnp.minimum(y2c, y2r)
    w = jnp.maximum(xx2 - xx1, 0.0)
    h = jnp.maximum(yy2 - yy1, 0.0)
    inter = w * h
    return inter / ((ac + ar) - inter)


def _nms_kernel(d_ref, out_ref, x1_ref, y1_ref, x2_ref, y2_ref, a_ref,
                supp_ref):
    def build(j, _):
        base = j * BLK
        blk = d_ref[pl.ds(base, BLK), :]
        x1c, y1c = blk[:, 0:1], blk[:, 1:2]
        x2c, y2c = blk[:, 2:3], blk[:, 3:4]
        x1_ref[0:1, pl.ds(base, BLK)] = _col_to_row(x1c)
        y1_ref[0:1, pl.ds(base, BLK)] = _col_to_row(y1c)
        x2_ref[0:1, pl.ds(base, BLK)] = _col_to_row(x2c)
        y2_ref[0:1, pl.ds(base, BLK)] = _col_to_row(y2c)
        a_ref[0:1, pl.ds(base, BLK)] = _col_to_row((x2c - x1c) * (y2c - y1c))
        supp_ref[0:1, pl.ds(base, BLK)] = jnp.zeros((1, BLK), F32)
        return 0

    lax.fori_loop(0, NB, build, 0)

    lid_r = lax.broadcasted_iota(jnp.int32, (BLK, BLK), 0)
    lid_c = lax.broadcasted_iota(jnp.int32, (BLK, BLK), 1)
    triu = (lid_c > lid_r).astype(F32)
    tril = (lid_c < lid_r).astype(F32)

    def outer(i, _):
        base = i * BLK
        blk = d_ref[pl.ds(base, BLK), :]
        x1c, y1c = blk[:, 0:1], blk[:, 1:2]
        x2c, y2c = blk[:, 2:3], blk[:, 3:4]
        ac = (x2c - x1c) * (y2c - y1c)
        rgid = base + lax.broadcasted_iota(jnp.int32, (BLK, 1), 0)

        x1r = x1_ref[0:1, pl.ds(base, BLK)]
        y1r = y1_ref[0:1, pl.ds(base, BLK)]
        x2r = x2_ref[0:1, pl.ds(base, BLK)]
        y2r = y2_ref[0:1, pl.ds(base, BLK)]
        ar = a_ref[0:1, pl.ds(base, BLK)]
        iou_ii = _iou(x1c, y1c, x2c, y2c, ac, x1r, y1r, x2r, y2r, ar)
        hitf = (iou_ii > THR).astype(F32)
        su = hitf * triu
        sl = hitf * tril
        alive_row = (supp_ref[0:1, pl.ds(base, BLK)] == 0.0).astype(F32)
        alive_col = _row_to_col(alive_row)

        def fp_step(keep_row):
            new_col = alive_col * (
                1.0 - jnp.max(sl * keep_row, axis=1, keepdims=True))
            new_row = alive_row * (
                1.0 - jnp.max(su * new_col, axis=0, keepdims=True))
            return new_row, new_col

        def fp_cond(carry):
            return carry[2]

        def fp_body(carry):
            keep_row, _, _ = carry
            new_row, new_col = fp_step(keep_row)
            return new_row, new_col, jnp.any(new_row != keep_row)

        r1, c1 = fp_step(alive_row)
        keep_row, keep_col, _ = lax.while_loop(
            fp_cond, fp_body, (r1, c1, jnp.any(r1 != alive_row)))
        supp_ref[0:1, pl.ds(base, BLK)] = 1.0 - keep_row
        out_ref[pl.ds(base, BLK), :] = blk[:, 0:16] * keep_col

        kept = keep_col > 0.0
        x1m = jnp.where(kept, x1c, 1.0e9)
        x2m = jnp.where(kept, x2c, -1.0e9)
        acm = ac * keep_col

        ci0 = i // (CHUNK // BLK)
        cs0 = pl.multiple_of(ci0 * CHUNK, CHUNK)

        def tail(cs, masked):
            x1t = x1_ref[0:1, pl.ds(cs, CHUNK)]
            y1t = y1_ref[0:1, pl.ds(cs, CHUNK)]
            x2t = x2_ref[0:1, pl.ds(cs, CHUNK)]
            y2t = y2_ref[0:1, pl.ds(cs, CHUNK)]
            at = a_ref[0:1, pl.ds(cs, CHUNK)]
            iou_t = _iou(x1m, y1c, x2m, y2c, acm, x1t, y1t, x2t, y2t, at)
            if masked:
                cgid = cs + lax.broadcasted_iota(jnp.int32, (1, CHUNK), 1)
                iou_t = jnp.where(cgid > rgid, iou_t, 0.0)
            sup = jnp.max(iou_t, axis=0, keepdims=True)
            supf = (sup > THR).astype(F32)
            old = supp_ref[0:1, pl.ds(cs, CHUNK)]
            supp_ref[0:1, pl.ds(cs, CHUNK)] = jnp.maximum(old, supf)

        tail(cs0, True)

        def chunk_body(ci, _):
            tail(pl.multiple_of(ci * CHUNK, CHUNK), False)
            return 0

        lax.fori_loop(ci0 + 1, NCHUNK, chunk_body, 0)
        return 0

    lax.fori_loop(0, NB, outer, 0)


def _nms_call(sorted_data):
    return pl.pallas_call(
        _nms_kernel,
        out_shape=jax.ShapeDtypeStruct((NP, 16), F32),
        scratch_shapes=[pltpu.VMEM((1, NP), F32)] * 6,
    )(sorted_data)


def kernel(boxes, scores):
    pad = NP - N
    far = -1.0e6
    pad_boxes = jnp.tile(
        jnp.array([[far, far, far + 1.0, far + 1.0]], F32), (pad, 1))
    b = jnp.concatenate([boxes.astype(F32), pad_boxes], axis=0)
    s = jnp.concatenate([scores.astype(F32), jnp.full((pad,), -1.0, F32)])
    data = jnp.concatenate(
        [b, s[:, None], jnp.zeros((NP, _D - 5), F32)], axis=1)
    rank = _rank_call(s[None, :])
    sorted_data = _gather_call_sc(data, rank)
    out16 = _nms_call(sorted_data[:, 0:16])
    return out16[:N, :5]


# speedup vs baseline: 1.0034x; 1.0034x over previous
import functools

import jax
import jax.numpy as jnp
from jax import lax
from jax.experimental import pallas as pl
from jax.experimental.pallas import tpu as pltpu
from jax.experimental.pallas import tpu_sc as plsc

N = 5000
NP = 5120
BLK = 128
NB = NP // BLK
CHUNK = 512
NCHUNK = NP // CHUNK
THR = 0.45
F32 = jnp.float32


def _ident():
    r = lax.broadcasted_iota(jnp.int32, (BLK, BLK), 0)
    c = lax.broadcasted_iota(jnp.int32, (BLK, BLK), 1)
    return (r == c).astype(F32)


def _row_to_col(v):
    return jnp.sum(_ident() * v, axis=1, keepdims=True)


def _col_to_row(u):
    return jnp.sum(_ident() * u, axis=0, keepdims=True)


def _rank_kernel(s_ref, rank_ref):
    srow = s_ref[0:1, :]
    cid = lax.broadcasted_iota(jnp.int32, (1, NP), 1)

    def body(bi, _):
        base = bi * BLK
        s_blk = s_ref[0:1, pl.ds(base, BLK)]
        s_col = _row_to_col(s_blk)
        rid = base + lax.broadcasted_iota(jnp.int32, (BLK, 1), 0)
        before = (srow > s_col) | ((srow == s_col) & (cid < rid))
        cnt = jnp.sum(before.astype(jnp.int32), axis=1, keepdims=True)
        rank_ref[pl.ds(base, BLK), :] = cnt
        return 0

    lax.fori_loop(0, NB, body, 0)


def _rank_call(s_row):
    return pl.pallas_call(
        _rank_kernel,
        out_shape=jax.ShapeDtypeStruct((NP, 1), jnp.int32),
    )(s_row)


_NC, _NS = 2, 16
_NW = _NC * _NS
_RPW = NP // _NW
_HALF = _RPW // 2
_D = 128


@functools.cache
def _make_scatter_sc():
    @functools.partial(
        pl.kernel,
        out_type=jax.ShapeDtypeStruct((NP, _D), F32),
        mesh=plsc.VectorSubcoreMesh(core_axis_name="c", subcore_axis_name="s"),
        scratch_types=[
            pltpu.VMEM((2, _HALF), jnp.int32),
            pltpu.VMEM((_RPW, _D), F32),
            pltpu.SemaphoreType.DMA,
        ],
    )
    def _scatter_sc(data_hbm, rank_hbm, out_hbm, idx_v, rows_v, sem):
        wid = lax.axis_index("s") * _NC + lax.axis_index("c")
        base = wid * _RPW
        pltpu.sync_copy(rank_hbm.at[wid], idx_v)
        pltpu.sync_copy(data_hbm.at[pl.ds(base, _RPW)], rows_v)
        for h in range(2):
            pltpu.async_copy(
                rows_v.at[pl.ds(h * _HALF, _HALF)],
                out_hbm.at[idx_v.at[h]],
                sem,
            ).wait()

    return _scatter_sc


def _gather_call_sc(data, rank):
    rank3 = rank[:, 0].reshape(_NW, 2, _HALF)
    return _make_scatter_sc()(data, rank3)


def _iou(x1c, y1c, x2c, y2c, ac, x1r, y1r, x2r, y2r, ar):
    xx1 = jnp.maximum(x1c, x1r)
    yy1 = jnp.maximum(y1c, y1r)
    xx2 = jnp.minimum(x2c, x2r)
    yy2 = jnp.minimum(y2c, y2r)
    w = jnp.maximum(xx2 - xx1, 0.0)
    h = jnp.maximum(yy2 - yy1, 0.0)
    inter = w * h
    return inter / ((ac + ar) - inter)


def _nms_kernel(d_ref, out_ref, x1_ref, y1_ref, x2_ref, y2_ref, a_ref,
                supp_ref):
    def build(j, _):
        base = j * BLK
        blk = d_ref[pl.ds(base, BLK), :]
        x1c, y1c = blk[:, 0:1], blk[:, 1:2]
        x2c, y2c = blk[:, 2:3], blk[:, 3:4]
        x1_ref[0:1, pl.ds(base, BLK)] = _col_to_row(x1c)
        y1_ref[0:1, pl.ds(base, BLK)] = _col_to_row(y1c)
        x2_ref[0:1, pl.ds(base, BLK)] = _col_to_row(x2c)
        y2_ref[0:1, pl.ds(base, BLK)] = _col_to_row(y2c)
        a_ref[0:1, pl.ds(base, BLK)] = _col_to_row((x2c - x1c) * (y2c - y1c))
        supp_ref[0:1, pl.ds(base, BLK)] = jnp.zeros((1, BLK), F32)
        return 0

    lax.fori_loop(0, NB, build, 0)

    lid_r = lax.broadcasted_iota(jnp.int32, (BLK, BLK), 0)
    lid_c = lax.broadcasted_iota(jnp.int32, (BLK, BLK), 1)
    triu = (lid_c > lid_r).astype(F32)
    tril = (lid_c < lid_r).astype(F32)

    def outer(i, _):
        base = i * BLK
        blk = d_ref[pl.ds(base, BLK), :]
        x1c, y1c = blk[:, 0:1], blk[:, 1:2]
        x2c, y2c = blk[:, 2:3], blk[:, 3:4]
        ac = (x2c - x1c) * (y2c - y1c)
        rgid = base + lax.broadcasted_iota(jnp.int32, (BLK, 1), 0)

        x1r = x1_ref[0:1, pl.ds(base, BLK)]
        y1r = y1_ref[0:1, pl.ds(base, BLK)]
        x2r = x2_ref[0:1, pl.ds(base, BLK)]
        y2r = y2_ref[0:1, pl.ds(base, BLK)]
        ar = a_ref[0:1, pl.ds(base, BLK)]
        iou_ii = _iou(x1c, y1c, x2c, y2c, ac, x1r, y1r, x2r, y2r, ar)
        hitf = (iou_ii > THR).astype(F32)
        su = hitf * triu
        sl = hitf * tril
        alive_row = (supp_ref[0:1, pl.ds(base, BLK)] == 0.0).astype(F32)
        alive_col = _row_to_col(alive_row)

        def fp_step(keep_row):
            new_col = alive_col * (
                1.0 - jnp.max(sl * keep_row, axis=1, keepdims=True))
            new_row = alive_row * (
                1.0 - jnp.max(su * new_col, axis=0, keepdims=True))
            return new_row, new_col

        def fp_cond(carry):
            return carry[2]

        def fp_body(carry):
            keep_row, _, _ = carry
            new_row, new_col = fp_step(keep_row)
            return new_row, new_col, jnp.any(new_row != keep_row)

        r1, c1 = fp_step(alive_row)
        keep_row, keep_col, _ = lax.while_loop(
            fp_cond, fp_body, (r1, c1, jnp.any(r1 != alive_row)))
        supp_ref[0:1, pl.ds(base, BLK)] = 1.0 - keep_row
        out_ref[pl.ds(base, BLK), :] = blk[:, 0:16] * keep_col

        kept = keep_col > 0.0
        x1m = jnp.where(kept, x1c, 1.0e9)
        x2m = jnp.where(kept, x2c, -1.0e9)
        acm = ac * keep_col

        ci0 = i // (CHUNK // BLK)
        cs0 = pl.multiple_of(ci0 * CHUNK, CHUNK)

        def tail(cs, masked):
            x1t = x1_ref[0:1, pl.ds(cs, CHUNK)]
            y1t = y1_ref[0:1, pl.ds(cs, CHUNK)]
            x2t = x2_ref[0:1, pl.ds(cs, CHUNK)]
            y2t = y2_ref[0:1, pl.ds(cs, CHUNK)]
            at = a_ref[0:1, pl.ds(cs, CHUNK)]
            iou_t = _iou(x1m, y1c, x2m, y2c, acm, x1t, y1t, x2t, y2t, at)
            if masked:
                cgid = cs + lax.broadcasted_iota(jnp.int32, (1, CHUNK), 1)
                iou_t = jnp.where(cgid > rgid, iou_t, 0.0)
            sup = jnp.max(iou_t, axis=0, keepdims=True)
            supf = (sup > THR).astype(F32)
            old = supp_ref[0:1, pl.ds(cs, CHUNK)]
            supp_ref[0:1, pl.ds(cs, CHUNK)] = jnp.maximum(old, supf)

        tail(cs0, True)

        def chunk_body(ci, _):
            tail(pl.multiple_of(ci * CHUNK, CHUNK), False)
            return 0

        lax.fori_loop(ci0 + 1, NCHUNK, chunk_body, 0)
        return 0

    lax.fori_loop(0, NB, outer, 0)


def _nms_call(sorted_data):
    return pl.pallas_call(
        _nms_kernel,
        out_shape=jax.ShapeDtypeStruct((NP, 16), F32),
        scratch_shapes=[pltpu.VMEM((1, NP), F32)] * 6,
    )(sorted_data)


def kernel(boxes, scores):
    pad = NP - N
    far = -1.0e6
    pad_boxes = jnp.tile(
        jnp.array([[far, far, far + 1.0, far + 1.0]], F32), (pad, 1))
    b = jnp.concatenate([boxes.astype(F32), pad_boxes], axis=0)
    s = jnp.concatenate([scores.astype(F32), jnp.full((pad,), -1.0, F32)])
    data = jnp.concatenate(
        [b, s[:, None], jnp.zeros((NP, _D - 5), F32)], axis=1)
    rank = _rank_call(s[None, :])
    sorted_data = _gather_call_sc(data, rank)
    out16 = _nms_call(sorted_data[:, 0:16])
    return out16[:N, :5]
